# baseline (speedup 1.0000x reference)
import functools
import math

import jax
import jax.numpy as jnp
from jax import lax
from jax.experimental import pallas as pl
from jax.experimental.pallas import tpu as pltpu
from jax.experimental.pallas import tpu_sc as plsc

NN = 50000
EE = 800000
HH = 64
DOUT = 128
NL = 3
NGAUSS = 50
NGRAPH = 64
RCUT = 5.0
NMETA = 11

NC, NS = 2, 16
HALFB = 25088
NPAD = 2 * HALFB
DUMMY = HALFB
ACC_ROWS = HALFB + 128
EPW = 25088
EPAD = 32 * EPW
ESPAN = EPAD // NS
ETILE = 1024
NTILE = ESPAN // ETILE
EB = 4096
NB = 1024

_F32 = jnp.float32
_I32 = jnp.int32


def _ssp(x):
    return jnp.log1p(jnp.exp(x)) - math.log(2.0)


def _d2_body(px_hbm, py_hbm, pz_hbm, src_hbm, dst_hbm, d2_hbm,
             table, srcv, dstv, accv):
    cid = lax.axis_index("c")
    sid = lax.axis_index("s")
    wid = sid * NC + cid
    base = wid * EPW
    for c, p_hbm in enumerate((px_hbm, py_hbm, pz_hbm)):
        pltpu.sync_copy(p_hbm, table)

        def tile_body(i, _, c=c):
            tb = base + i * 1568
            pltpu.sync_copy(src_hbm.at[pl.ds(tb, 1568)], srcv)
            pltpu.sync_copy(dst_hbm.at[pl.ds(tb, 1568)], dstv)

            def j_body(j, _):
                off = pl.multiple_of(j * 16, 16)
                s = plsc.load_gather(table, [srcv[pl.ds(off, 16)]])
                d = plsc.load_gather(table, [dstv[pl.ds(off, 16)]])
                e = s - d
                aoff = pl.multiple_of(i * 1568, 16) + off
                if c == 0:
                    accv[pl.ds(aoff, 16)] = e * e
                else:
                    accv[pl.ds(aoff, 16)] = accv[pl.ds(aoff, 16)] + e * e
                return 0

            lax.fori_loop(0, 1568 // 16, j_body, 0, unroll=4)
            return 0

        lax.fori_loop(0, EPW // 1568, tile_body, 0)
    pltpu.sync_copy(accv, d2_hbm.at[pl.ds(base, EPW)])


_d2_call = functools.partial(
    pl.kernel,
    out_type=jax.ShapeDtypeStruct((EPAD,), _F32),
    mesh=plsc.VectorSubcoreMesh(core_axis_name="c", subcore_axis_name="s"),
    scratch_types=[
        pltpu.VMEM((NPAD,), _F32),
        pltpu.VMEM((1568,), _I32),
        pltpu.VMEM((1568,), _I32),
        pltpu.VMEM((EPW,), _F32),
    ],
    compiler_params=pltpu.CompilerParams(needs_layout_passes=False),
)(_d2_body)


def _agg_body(hw_hbm, w_hbm, src2d_hbm, dst2d_hbm, agg_hbm,
              acc, srct, dstt, idxl, rows0, rows1, wv, sem, ssem):
    cid = lax.axis_index("c")
    sid = lax.axis_index("s")
    lo = cid * HALFB

    def zrow(r, _):
        for k in range(4):
            wv[r, pl.ds(k * 16, 16)] = jnp.zeros((16,), _F32)
        return 0

    lax.fori_loop(0, 128, zrow, 0, unroll=4)
    zb = sid * (ACC_ROWS // NS)
    for o in range(0, 1536, 128):
        pltpu.sync_copy(wv, acc.at[pl.ds(zb + o, 128)])
    pltpu.sync_copy(wv.at[pl.ds(0, 40)], acc.at[pl.ds(zb + 1536, 40)])
    plsc.subcore_barrier()

    bufs = (rows0, rows1)

    def tile(i, _):
        eb = sid * ESPAN + i * ETILE
        rb = sid * (ESPAN // 128) + i * 8
        pltpu.sync_copy(src2d_hbm.at[pl.ds(rb, 8)], srct)
        pltpu.sync_copy(dst2d_hbm.at[pl.ds(rb, 8)], dstt)
        pend = pltpu.async_copy(hw_hbm.at[srct.at[0]], bufs[0], sem)

        for j in range(8):
            def kb(k, _, j=j):
                off = pl.multiple_of(k * 16, 16)
                d16 = dstt[j, pl.ds(off, 16)]
                inb = (d16 >= lo) & (d16 < lo + HALFB)
                idxl[j, pl.ds(off, 16)] = jnp.where(inb, d16 - lo, DUMMY)
                return 0

            lax.fori_loop(0, 8, kb, 0, unroll=8)

        scat = None
        for j in range(8):
            cur = bufs[j % 2]
            pltpu.sync_copy(w_hbm.at[pl.ds(eb + j * 128, 128)], wv)
            pend.wait()
            if scat is not None:
                scat.wait()
            if j < 7:
                pend = pltpu.async_copy(hw_hbm.at[srct.at[j + 1]],
                                        bufs[(j + 1) % 2], sem)

            def mrow(r, _, cur=cur):
                for k in range(4):
                    cur[r, pl.ds(k * 16, 16)] = (
                        wv[r, pl.ds(k * 16, 16)] * cur[r, pl.ds(k * 16, 16)])
                return 0

            lax.fori_loop(0, 128, mrow, 0, unroll=4)
            scat = pltpu.async_copy(cur, acc.at[idxl.at[j]], ssem, add=True)
        scat.wait()
        return 0

    lax.fori_loop(0, NTILE, tile, 0)
    plsc.subcore_barrier()
    wb = sid * (HALFB // NS)
    pltpu.sync_copy(acc.at[pl.ds(wb, HALFB // NS)],
                    agg_hbm.at[pl.ds(lo + wb, HALFB // NS)])


_agg_call = functools.partial(
    pl.kernel,
    out_type=jax.ShapeDtypeStruct((NPAD, HH), _F32),
    mesh=plsc.VectorSubcoreMesh(core_axis_name="c", subcore_axis_name="s"),
    scratch_types=[
        pltpu.VMEM_SHARED((ACC_ROWS, HH), _F32),
        pltpu.VMEM((8, 128), _I32),
        pltpu.VMEM((8, 128), _I32),
        pltpu.VMEM((8, 128), _I32),
        pltpu.VMEM((128, HH), _F32),
        pltpu.VMEM((128, HH), _F32),
        pltpu.VMEM((128, HH), _F32),
        pltpu.SemaphoreType.DMA,
        pltpu.SemaphoreType.DMA,
    ],
    compiler_params=pltpu.CompilerParams(
        needs_layout_passes=False, use_tc_tiling_on_sc=False),
)(_agg_body)


def _edge_filters_body(d2_ref, w1c_ref, b1c_ref, w2_ref, b2_ref, out_ref):
    d2 = d2_ref[...]
    dist = jnp.sqrt(d2 + 1e-12)
    step = RCUT / (NGAUSS - 1)
    offs = lax.broadcasted_iota(_I32, (1, NGAUSS), 1).astype(_F32) * step
    coeff = -0.5 / step ** 2
    ea = jnp.exp(coeff * (dist - offs) ** 2)
    cc = 0.5 * (jnp.cos(dist * (math.pi / RCUT)) + 1.0)
    u = _ssp(jnp.dot(ea, w1c_ref[...], preferred_element_type=_F32)
             + b1c_ref[...])
    for t in range(NL):
        w = jnp.dot(u[:, t * HH:(t + 1) * HH], w2_ref[t],
                    preferred_element_type=_F32) + b2_ref[t][None, :]
        out_ref[t] = w * cc


def _edge_filters(d2, p):
    w1c = jnp.transpose(p['mlp_w1'], (1, 0, 2)).reshape(NGAUSS, NL * HH)
    b1c = p['mlp_b1'].reshape(1, NL * HH)
    return pl.pallas_call(
        _edge_filters_body,
        grid=(EPAD // EB,),
        in_specs=[
            pl.BlockSpec((EB, 1), lambda i: (i, 0)),
            pl.BlockSpec((NGAUSS, NL * HH), lambda i: (0, 0)),
            pl.BlockSpec((1, NL * HH), lambda i: (0, 0)),
            pl.BlockSpec((NL, HH, HH), lambda i: (0, 0, 0)),
            pl.BlockSpec((NL, HH), lambda i: (0, 0)),
        ],
        out_specs=pl.BlockSpec((NL, EB, HH), lambda i: (0, i, 0)),
        out_shape=jax.ShapeDtypeStruct((NL, EPAD, HH), _F32),
    )(d2.reshape(EPAD, 1), w1c, b1c, p['mlp_w2'], p['mlp_b2'])


def _init_body(z_ref, emb_ref, cw1_ref, h_ref, hw_ref):
    zb = z_ref[...]
    oh = (zb == lax.broadcasted_iota(_I32, (NB, 100), 1)).astype(_F32)
    h = jnp.dot(oh, emb_ref[...], preferred_element_type=_F32)
    h_ref[...] = h
    hw_ref[...] = jnp.dot(h, cw1_ref[0], preferred_element_type=_F32)


def _init(z_p, p):
    return pl.pallas_call(
        _init_body,
        grid=(NPAD // NB,),
        in_specs=[
            pl.BlockSpec((NB, 1), lambda i: (i, 0)),
            pl.BlockSpec((100, HH), lambda i: (0, 0)),
            pl.BlockSpec((NL, HH, HH), lambda i: (0, 0, 0)),
        ],
        out_specs=[
            pl.BlockSpec((NB, HH), lambda i: (i, 0)),
            pl.BlockSpec((NB, HH), lambda i: (i, 0)),
        ],
        out_shape=[
            jax.ShapeDtypeStruct((NPAD, HH), _F32),
            jax.ShapeDtypeStruct((NPAD, HH), _F32),
        ],
    )(z_p.reshape(NPAD, 1), p['emb'], p['conv_w1'])


def _update_body(h_ref, agg_ref, cw2_ref, cb2_ref, lw_ref, lb_ref, cw1n_ref,
                 h_out, hw_out):
    cv = _ssp(jnp.dot(agg_ref[...], cw2_ref[...],
                      preferred_element_type=_F32) + cb2_ref[...])
    hn = h_ref[...] + jnp.dot(cv, lw_ref[...],
                              preferred_element_type=_F32) + lb_ref[...]
    h_out[...] = hn
    hw_out[...] = jnp.dot(hn, cw1n_ref[...], preferred_element_type=_F32)


def _update(h, agg, cw2, cb2, lw, lb, cw1n):
    return pl.pallas_call(
        _update_body,
        grid=(NPAD // NB,),
        in_specs=[
            pl.BlockSpec((NB, HH), lambda i: (i, 0)),
            pl.BlockSpec((NB, HH), lambda i: (i, 0)),
            pl.BlockSpec((HH, HH), lambda i: (0, 0)),
            pl.BlockSpec((1, HH), lambda i: (0, 0)),
            pl.BlockSpec((HH, HH), lambda i: (0, 0)),
            pl.BlockSpec((1, HH), lambda i: (0, 0)),
            pl.BlockSpec((HH, HH), lambda i: (0, 0)),
        ],
        out_specs=[
            pl.BlockSpec((NB, HH), lambda i: (i, 0)),
            pl.BlockSpec((NB, HH), lambda i: (i, 0)),
        ],
        out_shape=[
            jax.ShapeDtypeStruct((NPAD, HH), _F32),
            jax.ShapeDtypeStruct((NPAD, HH), _F32),
        ],
    )(h, agg, cw2, cb2.reshape(1, HH), lw, lb.reshape(1, HH), cw1n)


def _readout_body(h_ref, batch_ref, w1_ref, b1_ref, w2_ref, b2_ref,
                  meta_ref, pw_ref, pb_ref, out_ref, acc_ref):
    i = pl.program_id(0)
    hb = _ssp(jnp.dot(h_ref[...], w1_ref[...],
                      preferred_element_type=_F32) + b1_ref[...])
    hb = jnp.dot(hb, w2_ref[...], preferred_element_type=_F32) + b2_ref[...]
    bb = batch_ref[...]
    oh = (bb == lax.broadcasted_iota(_I32, (NGRAPH, NB), 0)).astype(_F32)
    part = jnp.dot(oh, hb, preferred_element_type=_F32)

    @pl.when(i == 0)
    def _():
        acc_ref[...] = part

    @pl.when(i != 0)
    def _():
        acc_ref[...] = acc_ref[...] + part

    @pl.when(i == NPAD // NB - 1)
    def _():
        gm = jnp.concatenate([acc_ref[...], meta_ref[...]], axis=1)
        out_ref[...] = jnp.dot(gm, pw_ref[...],
                               preferred_element_type=_F32) + pb_ref[...]


def _readout(h, batch_p, meta, p):
    return pl.pallas_call(
        _readout_body,
        grid=(NPAD // NB,),
        in_specs=[
            pl.BlockSpec((NB, HH), lambda i: (i, 0)),
            pl.BlockSpec((1, NB), lambda i: (0, i)),
            pl.BlockSpec((HH, HH // 2), lambda i: (0, 0)),
            pl.BlockSpec((1, HH // 2), lambda i: (0, 0)),
            pl.BlockSpec((HH // 2, DOUT), lambda i: (0, 0)),
            pl.BlockSpec((1, DOUT), lambda i: (0, 0)),
            pl.BlockSpec((NGRAPH, NMETA), lambda i: (0, 0)),
            pl.BlockSpec((DOUT + NMETA, DOUT), lambda i: (0, 0)),
            pl.BlockSpec((1, DOUT), lambda i: (0, 0)),
        ],
        out_specs=pl.BlockSpec((NGRAPH, DOUT), lambda i: (0, 0)),
        out_shape=jax.ShapeDtypeStruct((NGRAPH, DOUT), _F32),
        scratch_shapes=[pltpu.VMEM((NGRAPH, DOUT), _F32)],
    )(h, batch_p.reshape(1, NPAD), p['out_w1'], p['out_b1'].reshape(1, HH // 2),
      p['out_w2'], p['out_b2'].reshape(1, DOUT), meta, p['proj_w'],
      p['proj_b'].reshape(1, DOUT))


def kernel(z, pos, batch, meta, edge_index, params):
    src = edge_index[0].astype(_I32)
    dst = edge_index[1].astype(_I32)
    src_p = jnp.concatenate([src, jnp.zeros((EPAD - EE,), _I32)])
    dst_p = jnp.concatenate([dst, jnp.full((EPAD - EE,), NPAD - 1, _I32)])
    pos_t = jnp.pad(pos.astype(_F32).T, ((0, 0), (0, NPAD - NN)))
    z_p = jnp.pad(z.astype(_I32), (0, NPAD - NN))
    batch_p = jnp.pad(batch.astype(_I32), (0, NPAD - NN), constant_values=127)
    src2d = src_p.reshape(EPAD // 128, 128)
    dst2d = dst_p.reshape(EPAD // 128, 128)

    d2 = _d2_call(pos_t[0], pos_t[1], pos_t[2], src_p, dst_p)
    w_all = _edge_filters(d2, params)
    h, hw = _init(z_p, params)
    for t in range(NL):
        agg = _agg_call(hw, w_all[t], src2d, dst2d)
        h, hw = _update(h, agg, params['conv_w2'][t], params['conv_b2'][t],
                        params['lin_w'][t], params['lin_b'][t],
                        params['conv_w1'][(t + 1) % NL])
    return _readout(h, batch_p, meta, params)


# speedup vs baseline: 1.0015x; 1.0015x over previous
import functools
import math

import jax
import jax.numpy as jnp
from jax import lax
from jax.experimental import pallas as pl
from jax.experimental.pallas import tpu as pltpu
from jax.experimental.pallas import tpu_sc as plsc

NN = 50000
EE = 800000
HH = 64
DOUT = 128
NL = 3
NGAUSS = 50
NGRAPH = 64
RCUT = 5.0
NMETA = 11

NC, NS = 2, 16
HALFB = 25088
NPAD = 2 * HALFB
DUMMY = HALFB
ACC_ROWS = HALFB + 128
EPW = 25088
EPAD = 32 * EPW
ESPAN = EPAD // NS
ETILE = 1024
NTILE = ESPAN // ETILE
EB = 8192
NB = 1024

_F32 = jnp.float32
_I32 = jnp.int32


def _ssp(x):
    return jnp.log1p(jnp.exp(x)) - math.log(2.0)


def _d2_body(px_hbm, py_hbm, pz_hbm, src_hbm, dst_hbm, d2_hbm,
             table, srcv, dstv, accv):
    cid = lax.axis_index("c")
    sid = lax.axis_index("s")
    wid = sid * NC + cid
    base = wid * EPW
    for c, p_hbm in enumerate((px_hbm, py_hbm, pz_hbm)):
        pltpu.sync_copy(p_hbm, table)

        def tile_body(i, _, c=c):
            tb = base + i * 1568
            pltpu.sync_copy(src_hbm.at[pl.ds(tb, 1568)], srcv)
            pltpu.sync_copy(dst_hbm.at[pl.ds(tb, 1568)], dstv)

            def j_body(j, _):
                off = pl.multiple_of(j * 16, 16)
                s = plsc.load_gather(table, [srcv[pl.ds(off, 16)]])
                d = plsc.load_gather(table, [dstv[pl.ds(off, 16)]])
                e = s - d
                aoff = pl.multiple_of(i * 1568, 16) + off
                if c == 0:
                    accv[pl.ds(aoff, 16)] = e * e
                else:
                    accv[pl.ds(aoff, 16)] = accv[pl.ds(aoff, 16)] + e * e
                return 0

            lax.fori_loop(0, 1568 // 16, j_body, 0, unroll=4)
            return 0

        lax.fori_loop(0, EPW // 1568, tile_body, 0)
    pltpu.sync_copy(accv, d2_hbm.at[pl.ds(base, EPW)])


_d2_call = functools.partial(
    pl.kernel,
    out_type=jax.ShapeDtypeStruct((EPAD,), _F32),
    mesh=plsc.VectorSubcoreMesh(core_axis_name="c", subcore_axis_name="s"),
    scratch_types=[
        pltpu.VMEM((NPAD,), _F32),
        pltpu.VMEM((1568,), _I32),
        pltpu.VMEM((1568,), _I32),
        pltpu.VMEM((EPW,), _F32),
    ],
    compiler_params=pltpu.CompilerParams(needs_layout_passes=False),
)(_d2_body)


def _agg_body(hw_hbm, w_hbm, src2d_hbm, dst2d_hbm, agg_hbm,
              acc, srct, dstt, idxl, rows0, rows1, wv, sem, ssem):
    cid = lax.axis_index("c")
    sid = lax.axis_index("s")
    lo = cid * HALFB

    def zrow(r, _):
        for k in range(4):
            wv[r, pl.ds(k * 16, 16)] = jnp.zeros((16,), _F32)
        return 0

    lax.fori_loop(0, 128, zrow, 0, unroll=4)
    zb = sid * (ACC_ROWS // NS)
    for o in range(0, 1536, 128):
        pltpu.sync_copy(wv, acc.at[pl.ds(zb + o, 128)])
    pltpu.sync_copy(wv.at[pl.ds(0, 40)], acc.at[pl.ds(zb + 1536, 40)])
    plsc.subcore_barrier()

    bufs = (rows0, rows1)

    def tile(i, _):
        eb = sid * ESPAN + i * ETILE
        rb = sid * (ESPAN // 128) + i * 8
        pltpu.sync_copy(src2d_hbm.at[pl.ds(rb, 8)], srct)
        pltpu.sync_copy(dst2d_hbm.at[pl.ds(rb, 8)], dstt)
        pend = pltpu.async_copy(hw_hbm.at[srct.at[0]], bufs[0], sem)

        for j in range(8):
            def kb(k, _, j=j):
                off = pl.multiple_of(k * 16, 16)
                d16 = dstt[j, pl.ds(off, 16)]
                inb = (d16 >= lo) & (d16 < lo + HALFB)
                idxl[j, pl.ds(off, 16)] = jnp.where(inb, d16 - lo, DUMMY)
                return 0

            lax.fori_loop(0, 8, kb, 0, unroll=8)

        scat = None
        for j in range(8):
            cur = bufs[j % 2]
            pltpu.sync_copy(w_hbm.at[pl.ds(eb + j * 128, 128)], wv)
            pend.wait()
            if scat is not None:
                scat.wait()
            if j < 7:
                pend = pltpu.async_copy(hw_hbm.at[srct.at[j + 1]],
                                        bufs[(j + 1) % 2], sem)

            def mrow(r, _, cur=cur):
                for k in range(4):
                    cur[r, pl.ds(k * 16, 16)] = (
                        wv[r, pl.ds(k * 16, 16)] * cur[r, pl.ds(k * 16, 16)])
                return 0

            lax.fori_loop(0, 128, mrow, 0, unroll=4)
            scat = pltpu.async_copy(cur, acc.at[idxl.at[j]], ssem, add=True)
        scat.wait()
        return 0

    lax.fori_loop(0, NTILE, tile, 0)
    plsc.subcore_barrier()
    wb = sid * (HALFB // NS)
    pltpu.sync_copy(acc.at[pl.ds(wb, HALFB // NS)],
                    agg_hbm.at[pl.ds(lo + wb, HALFB // NS)])


_agg_call = functools.partial(
    pl.kernel,
    out_type=jax.ShapeDtypeStruct((NPAD, HH), _F32),
    mesh=plsc.VectorSubcoreMesh(core_axis_name="c", subcore_axis_name="s"),
    scratch_types=[
        pltpu.VMEM_SHARED((ACC_ROWS, HH), _F32),
        pltpu.VMEM((8, 128), _I32),
        pltpu.VMEM((8, 128), _I32),
        pltpu.VMEM((8, 128), _I32),
        pltpu.VMEM((128, HH), _F32),
        pltpu.VMEM((128, HH), _F32),
        pltpu.VMEM((128, HH), _F32),
        pltpu.SemaphoreType.DMA,
        pltpu.SemaphoreType.DMA,
    ],
    compiler_params=pltpu.CompilerParams(
        needs_layout_passes=False, use_tc_tiling_on_sc=False),
)(_agg_body)


def _edge_filters_body(d2_ref, w1c_ref, b1c_ref, w2_ref, b2_ref, out_ref):
    d2 = d2_ref[...]
    dist = jnp.sqrt(d2 + 1e-12)
    step = RCUT / (NGAUSS - 1)
    offs = lax.broadcasted_iota(_I32, (1, NGAUSS), 1).astype(_F32) * step
    coeff = -0.5 / step ** 2
    ea = jnp.exp(coeff * (dist - offs) ** 2)
    cc = 0.5 * (jnp.cos(dist * (math.pi / RCUT)) + 1.0)
    u = _ssp(jnp.dot(ea, w1c_ref[...], preferred_element_type=_F32)
             + b1c_ref[...])
    for t in range(NL):
        w = jnp.dot(u[:, t * HH:(t + 1) * HH], w2_ref[t],
                    preferred_element_type=_F32) + b2_ref[t][None, :]
        out_ref[t] = w * cc


def _edge_filters(d2, p):
    w1c = jnp.transpose(p['mlp_w1'], (1, 0, 2)).reshape(NGAUSS, NL * HH)
    b1c = p['mlp_b1'].reshape(1, NL * HH)
    return pl.pallas_call(
        _edge_filters_body,
        grid=(EPAD // EB,),
        in_specs=[
            pl.BlockSpec((EB, 1), lambda i: (i, 0)),
            pl.BlockSpec((NGAUSS, NL * HH), lambda i: (0, 0)),
            pl.BlockSpec((1, NL * HH), lambda i: (0, 0)),
            pl.BlockSpec((NL, HH, HH), lambda i: (0, 0, 0)),
            pl.BlockSpec((NL, HH), lambda i: (0, 0)),
        ],
        out_specs=pl.BlockSpec((NL, EB, HH), lambda i: (0, i, 0)),
        out_shape=jax.ShapeDtypeStruct((NL, EPAD, HH), _F32),
        compiler_params=pltpu.CompilerParams(
            vmem_limit_bytes=100 * 1024 * 1024),
    )(d2.reshape(EPAD, 1), w1c, b1c, p['mlp_w2'], p['mlp_b2'])


def _init_body(z_ref, emb_ref, cw1_ref, h_ref, hw_ref):
    zb = z_ref[...]
    oh = (zb == lax.broadcasted_iota(_I32, (NB, 100), 1)).astype(_F32)
    h = jnp.dot(oh, emb_ref[...], preferred_element_type=_F32)
    h_ref[...] = h
    hw_ref[...] = jnp.dot(h, cw1_ref[0], preferred_element_type=_F32)


def _init(z_p, p):
    return pl.pallas_call(
        _init_body,
        grid=(NPAD // NB,),
        in_specs=[
            pl.BlockSpec((NB, 1), lambda i: (i, 0)),
            pl.BlockSpec((100, HH), lambda i: (0, 0)),
            pl.BlockSpec((NL, HH, HH), lambda i: (0, 0, 0)),
        ],
        out_specs=[
            pl.BlockSpec((NB, HH), lambda i: (i, 0)),
            pl.BlockSpec((NB, HH), lambda i: (i, 0)),
        ],
        out_shape=[
            jax.ShapeDtypeStruct((NPAD, HH), _F32),
            jax.ShapeDtypeStruct((NPAD, HH), _F32),
        ],
    )(z_p.reshape(NPAD, 1), p['emb'], p['conv_w1'])


def _update_body(h_ref, agg_ref, cw2_ref, cb2_ref, lw_ref, lb_ref, cw1n_ref,
                 h_out, hw_out):
    cv = _ssp(jnp.dot(agg_ref[...], cw2_ref[...],
                      preferred_element_type=_F32) + cb2_ref[...])
    hn = h_ref[...] + jnp.dot(cv, lw_ref[...],
                              preferred_element_type=_F32) + lb_ref[...]
    h_out[...] = hn
    hw_out[...] = jnp.dot(hn, cw1n_ref[...], preferred_element_type=_F32)


def _update(h, agg, cw2, cb2, lw, lb, cw1n):
    return pl.pallas_call(
        _update_body,
        grid=(NPAD // NB,),
        in_specs=[
            pl.BlockSpec((NB, HH), lambda i: (i, 0)),
            pl.BlockSpec((NB, HH), lambda i: (i, 0)),
            pl.BlockSpec((HH, HH), lambda i: (0, 0)),
            pl.BlockSpec((1, HH), lambda i: (0, 0)),
            pl.BlockSpec((HH, HH), lambda i: (0, 0)),
            pl.BlockSpec((1, HH), lambda i: (0, 0)),
            pl.BlockSpec((HH, HH), lambda i: (0, 0)),
        ],
        out_specs=[
            pl.BlockSpec((NB, HH), lambda i: (i, 0)),
            pl.BlockSpec((NB, HH), lambda i: (i, 0)),
        ],
        out_shape=[
            jax.ShapeDtypeStruct((NPAD, HH), _F32),
            jax.ShapeDtypeStruct((NPAD, HH), _F32),
        ],
    )(h, agg, cw2, cb2.reshape(1, HH), lw, lb.reshape(1, HH), cw1n)


def _readout_body(h_ref, batch_ref, w1_ref, b1_ref, w2_ref, b2_ref,
                  meta_ref, pw_ref, pb_ref, out_ref, acc_ref):
    i = pl.program_id(0)
    hb = _ssp(jnp.dot(h_ref[...], w1_ref[...],
                      preferred_element_type=_F32) + b1_ref[...])
    hb = jnp.dot(hb, w2_ref[...], preferred_element_type=_F32) + b2_ref[...]
    bb = batch_ref[...]
    oh = (bb == lax.broadcasted_iota(_I32, (NGRAPH, NB), 0)).astype(_F32)
    part = jnp.dot(oh, hb, preferred_element_type=_F32)

    @pl.when(i == 0)
    def _():
        acc_ref[...] = part

    @pl.when(i != 0)
    def _():
        acc_ref[...] = acc_ref[...] + part

    @pl.when(i == NPAD // NB - 1)
    def _():
        gm = jnp.concatenate([acc_ref[...], meta_ref[...]], axis=1)
        out_ref[...] = jnp.dot(gm, pw_ref[...],
                               preferred_element_type=_F32) + pb_ref[...]


def _readout(h, batch_p, meta, p):
    return pl.pallas_call(
        _readout_body,
        grid=(NPAD // NB,),
        in_specs=[
            pl.BlockSpec((NB, HH), lambda i: (i, 0)),
            pl.BlockSpec((1, NB), lambda i: (0, i)),
            pl.BlockSpec((HH, HH // 2), lambda i: (0, 0)),
            pl.BlockSpec((1, HH // 2), lambda i: (0, 0)),
            pl.BlockSpec((HH // 2, DOUT), lambda i: (0, 0)),
            pl.BlockSpec((1, DOUT), lambda i: (0, 0)),
            pl.BlockSpec((NGRAPH, NMETA), lambda i: (0, 0)),
            pl.BlockSpec((DOUT + NMETA, DOUT), lambda i: (0, 0)),
            pl.BlockSpec((1, DOUT), lambda i: (0, 0)),
        ],
        out_specs=pl.BlockSpec((NGRAPH, DOUT), lambda i: (0, 0)),
        out_shape=jax.ShapeDtypeStruct((NGRAPH, DOUT), _F32),
        scratch_shapes=[pltpu.VMEM((NGRAPH, DOUT), _F32)],
    )(h, batch_p.reshape(1, NPAD), p['out_w1'], p['out_b1'].reshape(1, HH // 2),
      p['out_w2'], p['out_b2'].reshape(1, DOUT), meta, p['proj_w'],
      p['proj_b'].reshape(1, DOUT))


def kernel(z, pos, batch, meta, edge_index, params):
    src = edge_index[0].astype(_I32)
    dst = edge_index[1].astype(_I32)
    src_p = jnp.concatenate([src, jnp.zeros((EPAD - EE,), _I32)])
    dst_p = jnp.concatenate([dst, jnp.full((EPAD - EE,), NPAD - 1, _I32)])
    pos_t = jnp.pad(pos.astype(_F32).T, ((0, 0), (0, NPAD - NN)))
    z_p = jnp.pad(z.astype(_I32), (0, NPAD - NN))
    batch_p = jnp.pad(batch.astype(_I32), (0, NPAD - NN), constant_values=127)
    src2d = src_p.reshape(EPAD // 128, 128)
    dst2d = dst_p.reshape(EPAD // 128, 128)

    d2 = _d2_call(pos_t[0], pos_t[1], pos_t[2], src_p, dst_p)
    w_all = _edge_filters(d2, params)
    h, hw = _init(z_p, params)
    for t in range(NL):
        agg = _agg_call(hw, w_all[t], src2d, dst2d)
        h, hw = _update(h, agg, params['conv_w2'][t], params['conv_b2'][t],
                        params['lin_w'][t], params['lin_b'][t],
                        params['conv_w1'][(t + 1) % NL])
    return _readout(h, batch_p, meta, params)
